# baseline (speedup 1.0000x reference)
import jax
import jax.numpy as jnp
from jax import lax
from jax.experimental import pallas as pl
from jax.experimental.pallas import tpu as pltpu
from jax.experimental.pallas import tpu_sc as plsc

V, K, C, F = 10000, 32, 4, 64
D = 128
NW = 32
RPW = V * K // NW
CH = 80
NCH = RPW // CH

VB = 80
VBK = VB * K
NG = VB // 8
H = 32


def _sc_gather_body(table_hbm, idx_hbm, out_hbm, idx_v, rows_v, sem):
    wid = lax.axis_index("s") * 2 + lax.axis_index("c")
    base = wid * RPW

    def body(i, carry):
        off = base + i * CH
        pltpu.sync_copy(idx_hbm.at[pl.ds(off, CH)], idx_v)
        pltpu.async_copy(table_hbm.at[idx_v], rows_v, sem).wait()
        pltpu.sync_copy(rows_v, out_hbm.at[pl.ds(off, CH)])
        return carry

    lax.fori_loop(0, NCH, body, 0)


def _sc_gather(table, idx):
    mesh = plsc.VectorSubcoreMesh(core_axis_name="c", subcore_axis_name="s")
    return pl.kernel(
        _sc_gather_body,
        mesh=mesh,
        out_type=jax.ShapeDtypeStruct((V * K, D), jnp.float32),
        scratch_types=[
            pltpu.VMEM((CH,), jnp.int32),
            pltpu.VMEM((CH, D), jnp.float32),
            pltpu.SemaphoreType.DMA,
        ],
    )(table, idx)


def _elu(x):
    return jnp.where(x > 0, x, jnp.exp(x) - 1.0)


def _tc_body(g_ref, d_ref, w1_ref, b1_ref, w2_ref, b2_ref, w3_ref, b3_ref,
             out_ref):
    g = g_ref[...]
    e = jnp.exp(-10.0 * d_ref[...])
    w = g[:, :F] * e
    cols = g[:, F:F + C]
    ones = jnp.ones((VBK, 1), jnp.float32)
    zeros3 = jnp.zeros((VBK, 3), jnp.float32)
    prods = [cols[:, c:c + 1] * cols for c in range(C)]
    p_pad = jnp.concatenate([ones, cols] + prods + [zeros3], axis=1)

    rs = lax.broadcasted_iota(jnp.int32, (256, 192), 0)
    cs = lax.broadcasted_iota(jnp.int32, (256, 192), 1)
    maskf = jnp.where((rs // 32) == (cs // 24), 1.0, 0.0)

    covs = []
    means = []
    for gi in range(NG):
        w8 = w[gi * 256:(gi + 1) * 256]
        pg = p_pad[gi * 256:(gi + 1) * 256]
        bd = jnp.concatenate([pg] * 8, axis=1) * maskf
        g8 = lax.dot_general(w8, bd, (((0,), (0,)), ((), ())),
                             preferred_element_type=jnp.float32)
        for v in range(8):
            b = 24 * v
            rw = 1.0 / (g8[:, b:b + 1] + 1e-4)
            mean = g8[:, b + 1:b + 5] * rw
            sxx = g8[:, b + 5:b + 21] * rw
            m2 = jnp.concatenate([mean[:, c:c + 1] * mean for c in range(C)],
                                 axis=1)
            covs.append(sxx - m2)
            means.append(mean)

    cov = jnp.concatenate(covs, axis=0)
    h1 = _elu(jnp.dot(cov, w1_ref[...],
                      preferred_element_type=jnp.float32) + b1_ref[...])
    h2 = _elu(jnp.dot(h1, w2_ref[...],
                      preferred_element_type=jnp.float32) + b2_ref[...])
    h3 = _elu(jnp.dot(h2, w3_ref[...],
                      preferred_element_type=jnp.float32) + b3_ref[...])

    x4 = h3.reshape(VB, F, H)
    xcat = jnp.concatenate([x4[:, f, :] for f in range(F)], axis=1)
    m4 = jnp.concatenate(means, axis=0).reshape(VB, F, C)
    mcat = jnp.concatenate([m4[:, f, :] for f in range(F)], axis=1)
    out_ref[...] = jnp.concatenate([xcat, mcat], axis=1)


def _tc_main(gathered, dflat, W1, b1, W2, b2, W3, b3):
    grid = V // VB
    return pl.pallas_call(
        _tc_body,
        grid=(grid,),
        in_specs=[
            pl.BlockSpec((VBK, D), lambda i: (i, 0)),
            pl.BlockSpec((VBK, 1), lambda i: (i, 0)),
            pl.BlockSpec((C * C, H), lambda i: (0, 0)),
            pl.BlockSpec((1, H), lambda i: (0, 0)),
            pl.BlockSpec((H, H), lambda i: (0, 0)),
            pl.BlockSpec((1, H), lambda i: (0, 0)),
            pl.BlockSpec((H, H), lambda i: (0, 0)),
            pl.BlockSpec((1, H), lambda i: (0, 0)),
        ],
        out_specs=pl.BlockSpec((VB, F * H + F * C), lambda i: (i, 0)),
        out_shape=jax.ShapeDtypeStruct((V, F * H + F * C), jnp.float32),
    )(gathered, dflat, W1, b1, W2, b2, W3, b3)


def kernel(coordinates, distsq, features, n_idxs, W1, b1, W2, b2, W3, b3):
    table = jnp.concatenate(
        [features, coordinates,
         jnp.zeros((V, D - F - C), jnp.float32)], axis=1)
    idx = n_idxs.reshape(-1).astype(jnp.int32)
    gathered = _sc_gather(table, idx)
    dflat = distsq.reshape(V * K, 1)
    return _tc_main(gathered, dflat, W1, b1.reshape(1, H), W2,
                    b2.reshape(1, H), W3, b3.reshape(1, H))


# speedup vs baseline: 2.4197x; 2.4197x over previous
import numpy as np
import jax
import jax.numpy as jnp
from jax import lax
from jax.experimental import pallas as pl
from jax.experimental.pallas import tpu as pltpu
from jax.experimental.pallas import tpu_sc as plsc

V, K, C, F = 10000, 32, 4, 64
D = 128
NW = 32
RPW = V * K // NW
CH = 80
NCH = RPW // CH

VB = 80
VBK = VB * K
NG = VB // 8
H = 32
NP = 21


def _sel_matrices():
    sa = np.zeros((D, NP), np.float32)
    sb = np.zeros((D, NP), np.float32)
    sa[F + C, 0] = 1.0
    sb[F + C, 0] = 1.0
    for c in range(C):
        sa[F + c, 1 + c] = 1.0
        sb[F + C, 1 + c] = 1.0
        for d in range(C):
            sa[F + c, 5 + 4 * c + d] = 1.0
            sb[F + d, 5 + 4 * c + d] = 1.0
    return jnp.asarray(sa), jnp.asarray(sb)


def _sc_gather_body(table_hbm, idx_hbm, out_hbm, idx_v, rows_v, sem):
    wid = lax.axis_index("s") * 2 + lax.axis_index("c")
    base = wid * RPW

    def body(i, carry):
        off = base + i * CH
        pltpu.sync_copy(idx_hbm.at[pl.ds(off, CH)], idx_v)
        pltpu.async_copy(table_hbm.at[idx_v], rows_v, sem).wait()
        pltpu.sync_copy(rows_v, out_hbm.at[pl.ds(off, CH)])
        return carry

    lax.fori_loop(0, NCH, body, 0)


def _sc_gather(table, idx):
    mesh = plsc.VectorSubcoreMesh(core_axis_name="c", subcore_axis_name="s")
    return pl.kernel(
        _sc_gather_body,
        mesh=mesh,
        out_type=jax.ShapeDtypeStruct((V * K, D), jnp.float32),
        scratch_types=[
            pltpu.VMEM((CH,), jnp.int32),
            pltpu.VMEM((CH, D), jnp.float32),
            pltpu.SemaphoreType.DMA,
        ],
    )(table, idx)


def _elu(x):
    return jnp.where(x > 0, x, jnp.exp(x) - 1.0)


def _tc_body(g_ref, d_ref, sa_ref, sb_ref, w1_ref, b1_ref, w2_ref, b2_ref,
             w3_ref, b3_ref, out_ref):
    g = g_ref[...]
    e = jnp.exp(-10.0 * d_ref[...])
    w = g[:, :F] * e
    pa = jnp.dot(g, sa_ref[...], preferred_element_type=jnp.float32)
    pb = jnp.dot(g, sb_ref[...], preferred_element_type=jnp.float32)
    p = pa * pb

    rs = lax.broadcasted_iota(jnp.int32, (256, 512), 0)
    cs = lax.broadcasted_iota(jnp.int32, (256, 512), 1)
    maskf = jnp.where((rs // K) == (cs // F), 1.0, 0.0)

    gs = []
    for gi in range(NG):
        w8 = w[gi * 256:(gi + 1) * 256]
        p8 = p[gi * 256:(gi + 1) * 256]
        wt = jnp.concatenate([w8] * 8, axis=1) * maskf
        gs.append(lax.dot_general(wt, p8, (((0,), (0,)), ((), ())),
                                  preferred_element_type=jnp.float32))
    gmat = jnp.concatenate(gs, axis=0)

    rw = 1.0 / (gmat[:, 0:1] + 1e-4)
    mean = gmat[:, 1:5] * rw
    sxx = gmat[:, 5:21] * rw
    m2 = jnp.concatenate([mean[:, c:c + 1] * mean for c in range(C)], axis=1)
    cov = sxx - m2

    h1 = _elu(jnp.dot(cov, w1_ref[...],
                      preferred_element_type=jnp.float32) + b1_ref[...])
    h2 = _elu(jnp.dot(h1, w2_ref[...],
                      preferred_element_type=jnp.float32) + b2_ref[...])
    h3 = _elu(jnp.dot(h2, w3_ref[...],
                      preferred_element_type=jnp.float32) + b3_ref[...])

    x4 = h3.reshape(VB, F, H)
    xcat = jnp.concatenate([x4[:, f, :] for f in range(F)], axis=1)
    m4 = mean.reshape(VB, F, C)
    mcat = jnp.concatenate([m4[:, f, :] for f in range(F)], axis=1)
    out_ref[...] = jnp.concatenate([xcat, mcat], axis=1)


def _tc_main(gathered, dflat, sa, sb, W1, b1, W2, b2, W3, b3):
    grid = V // VB
    return pl.pallas_call(
        _tc_body,
        grid=(grid,),
        in_specs=[
            pl.BlockSpec((VBK, D), lambda i: (i, 0)),
            pl.BlockSpec((VBK, 1), lambda i: (i, 0)),
            pl.BlockSpec((D, NP), lambda i: (0, 0)),
            pl.BlockSpec((D, NP), lambda i: (0, 0)),
            pl.BlockSpec((C * C, H), lambda i: (0, 0)),
            pl.BlockSpec((1, H), lambda i: (0, 0)),
            pl.BlockSpec((H, H), lambda i: (0, 0)),
            pl.BlockSpec((1, H), lambda i: (0, 0)),
            pl.BlockSpec((H, H), lambda i: (0, 0)),
            pl.BlockSpec((1, H), lambda i: (0, 0)),
        ],
        out_specs=pl.BlockSpec((VB, F * H + F * C), lambda i: (i, 0)),
        out_shape=jax.ShapeDtypeStruct((V, F * H + F * C), jnp.float32),
    )(gathered, dflat, sa, sb, W1, b1, W2, b2, W3, b3)


def kernel(coordinates, distsq, features, n_idxs, W1, b1, W2, b2, W3, b3):
    table = jnp.concatenate(
        [features, coordinates, jnp.ones((V, 1), jnp.float32),
         jnp.zeros((V, D - F - C - 1), jnp.float32)], axis=1)
    idx = n_idxs.reshape(-1).astype(jnp.int32)
    gathered = _sc_gather(table, idx)
    dflat = distsq.reshape(V * K, 1)
    sa, sb = _sel_matrices()
    return _tc_main(gathered, dflat, sa, sb, W1, b1.reshape(1, H), W2,
                    b2.reshape(1, H), W3, b3.reshape(1, H))


# speedup vs baseline: 3.1956x; 1.3207x over previous
import numpy as np
import jax
import jax.numpy as jnp
from jax import lax
from jax.experimental import pallas as pl
from jax.experimental.pallas import tpu as pltpu
from jax.experimental.pallas import tpu_sc as plsc

V, K, C, F = 10000, 32, 4, 64
D = 128
NW = 32
RPW = V * K // NW
CH = 80
NCH = RPW // CH

VB = 80
VBK = VB * K
NG = VB // 8
H = 32
NP = 21


def _sel_matrices():
    sa = np.zeros((D, NP), np.float32)
    sb = np.zeros((D, NP), np.float32)
    sa[F + C, 0] = 1.0
    sb[F + C, 0] = 1.0
    for c in range(C):
        sa[F + c, 1 + c] = 1.0
        sb[F + C, 1 + c] = 1.0
        for d in range(C):
            sa[F + c, 5 + 4 * c + d] = 1.0
            sb[F + d, 5 + 4 * c + d] = 1.0
    ma = np.zeros((NP, C * C), np.float32)
    mb = np.zeros((NP, C * C), np.float32)
    for c in range(C):
        for d in range(C):
            ma[1 + c, 4 * c + d] = 1.0
            mb[1 + d, 4 * c + d] = 1.0
    return jnp.asarray(sa), jnp.asarray(sb), jnp.asarray(ma), jnp.asarray(mb)


def _sc_gather_body(table_hbm, idx_hbm, out_hbm, idx_v, rows_v, sem):
    wid = lax.axis_index("s") * 2 + lax.axis_index("c")
    base = wid * RPW

    def body(i, carry):
        off = base + i * CH
        pltpu.sync_copy(idx_hbm.at[pl.ds(off, CH)], idx_v)
        pltpu.async_copy(table_hbm.at[idx_v], rows_v, sem).wait()
        pltpu.sync_copy(rows_v, out_hbm.at[pl.ds(off, CH)])
        return carry

    lax.fori_loop(0, NCH, body, 0)


def _sc_gather(table, idx):
    mesh = plsc.VectorSubcoreMesh(core_axis_name="c", subcore_axis_name="s")
    return pl.kernel(
        _sc_gather_body,
        mesh=mesh,
        out_type=jax.ShapeDtypeStruct((V * K, D), jnp.float32),
        scratch_types=[
            pltpu.VMEM((CH,), jnp.int32),
            pltpu.VMEM((CH, D), jnp.float32),
            pltpu.SemaphoreType.DMA,
        ],
    )(table, idx)


def _elu(x):
    return jnp.where(x > 0, x, jnp.exp(x) - 1.0)


def _tc_body(g_ref, d_ref, sa_ref, sb_ref, ma_ref, mb_ref, w1_ref, b1_ref,
             w2_ref, b2_ref, w3_ref, b3_ref, out_ref):
    g = g_ref[...]
    e = jnp.exp(-10.0 * d_ref[...])
    w = g[:, :F] * e
    pa = jnp.dot(g, sa_ref[...], preferred_element_type=jnp.float32)
    pb = jnp.dot(g, sb_ref[...], preferred_element_type=jnp.float32)
    p = pa * pb

    rs = lax.broadcasted_iota(jnp.int32, (256, 512), 0)
    cs = lax.broadcasted_iota(jnp.int32, (256, 512), 1)
    maskf = jnp.where((rs // K) == (cs // F), 1.0, 0.0)

    gs = []
    for gi in range(NG):
        w8 = w[gi * 256:(gi + 1) * 256]
        p8 = p[gi * 256:(gi + 1) * 256]
        wt = jnp.concatenate([w8] * 8, axis=1) * maskf
        gs.append(lax.dot_general(wt, p8, (((0,), (0,)), ((), ())),
                                  preferred_element_type=jnp.float32))
    gmat = jnp.concatenate(gs, axis=0)

    rw = 1.0 / (gmat[:, 0:1] + 1e-4)
    mean = gmat[:, 1:5] * rw
    ga = jnp.dot(gmat, ma_ref[...], preferred_element_type=jnp.float32)
    gb = jnp.dot(gmat, mb_ref[...], preferred_element_type=jnp.float32)
    cov = (gmat[:, 5:21] - ga * gb * rw) * rw

    h1 = _elu(jnp.dot(cov, w1_ref[...],
                      preferred_element_type=jnp.float32) + b1_ref[...])
    h2 = _elu(jnp.dot(h1, w2_ref[...],
                      preferred_element_type=jnp.float32) + b2_ref[...])
    h3 = _elu(jnp.dot(h2, w3_ref[...],
                      preferred_element_type=jnp.float32) + b3_ref[...])

    x4 = h3.reshape(VB, F, H)
    xcat = jnp.concatenate([x4[:, f, :] for f in range(F)], axis=1)
    m4 = mean.reshape(VB, F, C)
    mcat = jnp.concatenate([m4[:, f, :] for f in range(F)], axis=1)
    out_ref[...] = jnp.concatenate([xcat, mcat], axis=1)


def _tc_main(gathered, dflat, sa, sb, ma, mb, W1, b1, W2, b2, W3, b3):
    grid = V // VB
    return pl.pallas_call(
        _tc_body,
        grid=(grid,),
        in_specs=[
            pl.BlockSpec((VBK, D), lambda i: (i, 0)),
            pl.BlockSpec((VBK, 1), lambda i: (i, 0)),
            pl.BlockSpec((D, NP), lambda i: (0, 0)),
            pl.BlockSpec((D, NP), lambda i: (0, 0)),
            pl.BlockSpec((NP, C * C), lambda i: (0, 0)),
            pl.BlockSpec((NP, C * C), lambda i: (0, 0)),
            pl.BlockSpec((C * C, H), lambda i: (0, 0)),
            pl.BlockSpec((1, H), lambda i: (0, 0)),
            pl.BlockSpec((H, H), lambda i: (0, 0)),
            pl.BlockSpec((1, H), lambda i: (0, 0)),
            pl.BlockSpec((H, H), lambda i: (0, 0)),
            pl.BlockSpec((1, H), lambda i: (0, 0)),
        ],
        out_specs=pl.BlockSpec((VB, F * H + F * C), lambda i: (i, 0)),
        out_shape=jax.ShapeDtypeStruct((V, F * H + F * C), jnp.float32),
    )(gathered, dflat, sa, sb, ma, mb, W1, b1, W2, b2, W3, b3)


def kernel(coordinates, distsq, features, n_idxs, W1, b1, W2, b2, W3, b3):
    table = jnp.concatenate(
        [features, coordinates, jnp.ones((V, 1), jnp.float32),
         jnp.zeros((V, D - F - C - 1), jnp.float32)], axis=1)
    idx = n_idxs.reshape(-1).astype(jnp.int32)
    gathered = _sc_gather(table, idx)
    dflat = distsq.reshape(V * K, 1)
    sa, sb, ma, mb = _sel_matrices()
    return _tc_main(gathered, dflat, sa, sb, ma, mb, W1, b1.reshape(1, H), W2,
                    b2.reshape(1, H), W3, b3.reshape(1, H))
